# baseline (speedup 1.0000x reference)
import functools

import jax
import jax.numpy as jnp
from jax import lax
from jax.experimental import pallas as pl
from jax.experimental.pallas import tpu as pltpu
from jax.experimental.pallas import tpu_sc as plsc

N = 10000
E = 320000
H = 128
HH = H // 2
K = 5

NCORES = 2
NSUB = 16
EPT = E // NSUB
CH2 = 256
NCH = 80
EPAD = NCH * CH2
EL = E + EPAD - EPT
NPAD = 10112
RPT = NPAD // NSUB
ZQ = 8
ZROWS = RPT // ZQ


def _make_segsum_body(table_half_rows, mode):
    off = table_half_rows

    def body(table, srcs, dsts, out, src_v, dst_v, src_cur0, src_cur1,
             rows0, rows1, zbuf, y_sh, gsem0, gsem1):
        c = lax.axis_index("c")
        s = lax.axis_index("s")
        coff = c.astype(jnp.int32) * off

        if mode == "gather":
            pltpu.sync_copy(srcs.at[s], src_v)
        pltpu.sync_copy(dsts.at[s], dst_v)

        @pl.loop(0, ZROWS)
        def _zero(i):
            for j in range(HH // 16):
                zbuf[i, pl.ds(j * 16, 16)] = jnp.zeros((16,), jnp.float32)

        for q in range(ZQ):
            pltpu.sync_copy(zbuf, y_sh.at[pl.ds(s * RPT + q * ZROWS, ZROWS)])

        if mode == "ones":
            @pl.loop(0, CH2)
            def _fill(i):
                for j in range(HH // 16):
                    rows0[i, pl.ds(j * 16, 16)] = (
                        jnp.zeros((16,), jnp.float32) + 1.0)

        plsc.subcore_barrier()

        def scat(j, rows):
            pltpu.sync_copy(rows, y_sh.at[dst_v.at[j]], add=True)

        if mode == "ones":
            @pl.loop(0, NCH)
            def _edges(j):
                scat(j, rows0)
        else:
            lbase = (c * EL + s * EPT) if mode == "linear" else 0

            def fetch_src(j, sc):
                if mode == "linear":
                    return table.at[pl.ds(lbase + j * CH2, CH2)]
                return table.at[sc]

            def fire(j, sc, rows, sem):
                if mode == "gather":
                    for i in range(CH2 // 16):
                        sc[pl.ds(i * 16, 16)] = (
                            src_v[j, pl.ds(i * 16, 16)] + coff)
                pltpu.async_copy(fetch_src(j, sc), rows, sem)

            def wait(j, sc, rows, sem):
                pltpu.make_async_copy(fetch_src(j, sc), rows, sem).wait()

            fire(0, src_cur0, rows0, gsem0)
            fire(1, src_cur1, rows1, gsem1)

            @pl.loop(0, NCH // 2 - 1)
            def _pairs(g):
                j = 2 * g
                wait(j, src_cur0, rows0, gsem0)
                scat(j, rows0)
                fire(j + 2, src_cur0, rows0, gsem0)
                wait(j + 1, src_cur1, rows1, gsem1)
                scat(j + 1, rows1)
                fire(j + 3, src_cur1, rows1, gsem1)

            wait(NCH - 2, src_cur0, rows0, gsem0)
            scat(NCH - 2, rows0)
            wait(NCH - 1, src_cur1, rows1, gsem1)
            scat(NCH - 1, rows1)

        plsc.subcore_barrier()

        pltpu.sync_copy(y_sh.at[pl.ds(s * RPT, RPT)],
                        out.at[c, pl.ds(s * RPT, RPT)])

    return body


@functools.partial(jax.jit, static_argnums=(3, 4))
def _sc_segsum(table, srcs, dsts, table_half_rows, mode="gather"):
    mesh = plsc.VectorSubcoreMesh(core_axis_name="c", subcore_axis_name="s",
                                  num_cores=NCORES, num_subcores=NSUB)
    f = pl.kernel(
        _make_segsum_body(table_half_rows, mode),
        out_type=jax.ShapeDtypeStruct((NCORES, NPAD, HH), jnp.float32),
        mesh=mesh,
        scratch_types=[
            pltpu.VMEM((NCH, CH2), jnp.int32),
            pltpu.VMEM((NCH, CH2), jnp.int32),
            pltpu.VMEM((CH2,), jnp.int32),
            pltpu.VMEM((CH2,), jnp.int32),
            pltpu.VMEM((CH2, HH), jnp.float32),
            pltpu.VMEM((CH2, HH), jnp.float32),
            pltpu.VMEM((ZROWS, HH), jnp.float32),
            pltpu.VMEM_SHARED((NPAD, HH), jnp.float32),
            pltpu.SemaphoreType.DMA,
            pltpu.SemaphoreType.DMA,
        ],
        compiler_params=pltpu.CompilerParams(use_tc_tiling_on_sc=False),
    )
    return f(table, srcs, dsts)


def _pad_idx(idx, pad_value):
    idx = idx.reshape(NSUB, EPT)
    pad = jnp.full((NSUB, EPAD - EPT), pad_value, jnp.int32)
    return jnp.concatenate([idx, pad], axis=1).reshape(NSUB, NCH, CH2)


def _bn_relu(x, g, b):
    m = x.mean(axis=0)
    v = x.var(axis=0)
    return jax.nn.relu((x - m) / jnp.sqrt(v + 1e-5) * g + b)


def _tagconv(h, srcs, dsts, norm, n2pad, w, b):
    t = (h * norm).reshape(N, 2, HH).swapaxes(0, 1).reshape(2 * N, HH)
    acc = h @ w[:H]
    for k in range(1, K + 1):
        parts = _sc_segsum(t, srcs, dsts, N if k == 1 else NPAD)
        wk = w[k * H:(k + 1) * H]
        pk = norm * (parts[0, :N] @ wk[:HH] + parts[1, :N] @ wk[HH:])
        acc = acc + pk
        if k < K:
            t = (parts * n2pad).reshape(2 * NPAD, HH)
    return acc + b


def kernel(node_features, edge_index, edge_attr, lap_pe, params):
    src = edge_index[0]
    dst = edge_index[1]
    srcs = _pad_idx(src, 0)
    dsts = _pad_idx(dst, N)

    ones_tab = jnp.ones((16, HH), jnp.float32)
    deg = _sc_segsum(ones_tab, srcs, dsts, 8, "ones")
    deg = deg[0, :N, 0]
    norm = jnp.power(jnp.clip(deg, 1.0, None), -0.5)[:, None]
    n2pad = jnp.pad((norm * norm), ((0, NPAD - N), (0, 0)))[None]

    eproc = edge_attr @ params["edge_w"] + params["edge_b"]
    et = eproc.reshape(E, 2, HH).swapaxes(0, 1)
    et = jnp.pad(et, ((0, 0), (0, EL - E), (0, 0))).reshape(2 * EL, HH)
    agg = _sc_segsum(et, srcs, dsts, E, "linear")
    agg_edge = jnp.concatenate([agg[0, :N], agg[1, :N]], axis=1)

    h = (node_features @ params["in_w"] + params["in_b"]
         + lap_pe @ params["pos_w"] + params["pos_b"]
         + agg_edge)
    for m in params["blocks"]:
        h_in = h
        h = _tagconv(h, srcs, dsts, norm, n2pad, m["tag1_w"], m["tag1_b"])
        h = _bn_relu(h, m["bn1_g"], m["bn1_b"])
        h = _tagconv(h, srcs, dsts, norm, n2pad, m["tag2_w"], m["tag2_b"])
        h = _bn_relu(h, m["bn2_g"], m["bn2_b"])
        h = h @ m["ff_w"] + m["ff_b"]
        h = h + h_in
    pose = (jax.nn.relu(h @ params["pose1_w"] + params["pose1_b"])
            @ params["pose2_w"] + params["pose2_b"])
    y = h.mean(axis=0, keepdims=True)
    label = (jax.nn.relu(y @ params["lab1_w"] + params["lab1_b"])
             @ params["lab2_w"] + params["lab2_b"])
    return (pose, label)


# speedup vs baseline: 1.2558x; 1.2558x over previous
import functools

import jax
import jax.numpy as jnp
from jax import lax
from jax.experimental import pallas as pl
from jax.experimental.pallas import tpu as pltpu
from jax.experimental.pallas import tpu_sc as plsc

N = 10000
E = 320000
H = 128
HH = H // 2
K = 5

NCORES = 2
NSUB = 16
EPT = E // NSUB
CH2 = 128
NCH = 158
EPAD = NCH * CH2
EL = E + EPAD - EPT
NPAD = 10112
RPT = NPAD // NSUB
ZQ = 8
ZROWS = RPT // ZQ


def _make_segsum_body(table_half_rows, mode):
    off = table_half_rows

    def body(table, srcs, dsts, out, src_v, dst_v, src_cur0, src_cur1,
             rows0, rows1, zbuf, y_sh, gsem0, gsem1):
        c = lax.axis_index("c")
        s = lax.axis_index("s")
        coff = c.astype(jnp.int32) * off

        if mode == "gather":
            pltpu.sync_copy(srcs.at[s], src_v)
        pltpu.sync_copy(dsts.at[s], dst_v)

        @pl.loop(0, ZROWS)
        def _zero(i):
            for j in range(HH // 16):
                zbuf[i, pl.ds(j * 16, 16)] = jnp.zeros((16,), jnp.float32)

        for q in range(ZQ):
            pltpu.sync_copy(zbuf, y_sh.at[pl.ds(s * RPT + q * ZROWS, ZROWS)])

        if mode == "ones":
            @pl.loop(0, CH2)
            def _fill(i):
                for j in range(HH // 16):
                    rows0[i, pl.ds(j * 16, 16)] = (
                        jnp.zeros((16,), jnp.float32) + 1.0)

        plsc.subcore_barrier()

        def scat(j, rows):
            pltpu.sync_copy(rows, y_sh.at[dst_v.at[j]], add=True)

        if mode == "ones":
            @pl.loop(0, NCH)
            def _edges(j):
                scat(j, rows0)
        else:
            lbase = (c * EL + s * EPT) if mode == "linear" else 0

            def fetch_src(j, sc):
                if mode == "linear":
                    return table.at[pl.ds(lbase + j * CH2, CH2)]
                return table.at[sc]

            def fire(j, sc, rows, sem):
                if mode == "gather":
                    for i in range(CH2 // 16):
                        sc[pl.ds(i * 16, 16)] = (
                            src_v[j, pl.ds(i * 16, 16)] + coff)
                pltpu.async_copy(fetch_src(j, sc), rows, sem)

            def wait(j, sc, rows, sem):
                pltpu.make_async_copy(fetch_src(j, sc), rows, sem).wait()

            fire(0, src_cur0, rows0, gsem0)
            fire(1, src_cur1, rows1, gsem1)

            @pl.loop(0, NCH // 2 - 1)
            def _pairs(g):
                j = 2 * g
                wait(j, src_cur0, rows0, gsem0)
                scat(j, rows0)
                fire(j + 2, src_cur0, rows0, gsem0)
                wait(j + 1, src_cur1, rows1, gsem1)
                scat(j + 1, rows1)
                fire(j + 3, src_cur1, rows1, gsem1)

            wait(NCH - 2, src_cur0, rows0, gsem0)
            scat(NCH - 2, rows0)
            wait(NCH - 1, src_cur1, rows1, gsem1)
            scat(NCH - 1, rows1)

        plsc.subcore_barrier()

        pltpu.sync_copy(y_sh.at[pl.ds(s * RPT, RPT)],
                        out.at[c, pl.ds(s * RPT, RPT)])

    return body


@functools.partial(jax.jit, static_argnums=(3, 4))
def _sc_segsum(table, srcs, dsts, table_half_rows, mode="gather"):
    mesh = plsc.VectorSubcoreMesh(core_axis_name="c", subcore_axis_name="s",
                                  num_cores=NCORES, num_subcores=NSUB)
    f = pl.kernel(
        _make_segsum_body(table_half_rows, mode),
        out_type=jax.ShapeDtypeStruct((NCORES, NPAD, HH), jnp.float32),
        mesh=mesh,
        scratch_types=[
            pltpu.VMEM((NCH, CH2), jnp.int32),
            pltpu.VMEM((NCH, CH2), jnp.int32),
            pltpu.VMEM((CH2,), jnp.int32),
            pltpu.VMEM((CH2,), jnp.int32),
            pltpu.VMEM((CH2, HH), jnp.float32),
            pltpu.VMEM((CH2, HH), jnp.float32),
            pltpu.VMEM((ZROWS, HH), jnp.float32),
            pltpu.VMEM_SHARED((NPAD, HH), jnp.float32),
            pltpu.SemaphoreType.DMA,
            pltpu.SemaphoreType.DMA,
        ],
        compiler_params=pltpu.CompilerParams(use_tc_tiling_on_sc=False),
    )
    return f(table, srcs, dsts)


def _pad_idx(idx, pad_value):
    idx = idx.reshape(NSUB, EPT)
    pad = jnp.full((NSUB, EPAD - EPT), pad_value, jnp.int32)
    return jnp.concatenate([idx, pad], axis=1).reshape(NSUB, NCH, CH2)


def _bn_relu(x, g, b):
    m = x.mean(axis=0)
    v = x.var(axis=0)
    return jax.nn.relu((x - m) / jnp.sqrt(v + 1e-5) * g + b)


def _tagconv(h, srcs, dsts, norm, n2pad, w, b):
    t = (h * norm).reshape(N, 2, HH).swapaxes(0, 1).reshape(2 * N, HH)
    acc = h @ w[:H]
    for k in range(1, K + 1):
        parts = _sc_segsum(t, srcs, dsts, N if k == 1 else NPAD)
        wk = w[k * H:(k + 1) * H]
        pk = norm * (parts[0, :N] @ wk[:HH] + parts[1, :N] @ wk[HH:])
        acc = acc + pk
        if k < K:
            t = (parts * n2pad).reshape(2 * NPAD, HH)
    return acc + b


def kernel(node_features, edge_index, edge_attr, lap_pe, params):
    src = edge_index[0]
    dst = edge_index[1]
    srcs = _pad_idx(src, 0)
    dsts = _pad_idx(dst, N)

    ones_tab = jnp.ones((16, HH), jnp.float32)
    deg = _sc_segsum(ones_tab, srcs, dsts, 8, "ones")
    deg = deg[0, :N, 0]
    norm = jnp.power(jnp.clip(deg, 1.0, None), -0.5)[:, None]
    n2pad = jnp.pad((norm * norm), ((0, NPAD - N), (0, 0)))[None]

    eproc = edge_attr @ params["edge_w"] + params["edge_b"]
    et = eproc.reshape(E, 2, HH).swapaxes(0, 1)
    et = jnp.pad(et, ((0, 0), (0, EL - E), (0, 0))).reshape(2 * EL, HH)
    agg = _sc_segsum(et, srcs, dsts, E, "linear")
    agg_edge = jnp.concatenate([agg[0, :N], agg[1, :N]], axis=1)

    h = (node_features @ params["in_w"] + params["in_b"]
         + lap_pe @ params["pos_w"] + params["pos_b"]
         + agg_edge)
    for m in params["blocks"]:
        h_in = h
        h = _tagconv(h, srcs, dsts, norm, n2pad, m["tag1_w"], m["tag1_b"])
        h = _bn_relu(h, m["bn1_g"], m["bn1_b"])
        h = _tagconv(h, srcs, dsts, norm, n2pad, m["tag2_w"], m["tag2_b"])
        h = _bn_relu(h, m["bn2_g"], m["bn2_b"])
        h = h @ m["ff_w"] + m["ff_b"]
        h = h + h_in
    pose = (jax.nn.relu(h @ params["pose1_w"] + params["pose1_b"])
            @ params["pose2_w"] + params["pose2_b"])
    y = h.mean(axis=0, keepdims=True)
    label = (jax.nn.relu(y @ params["lab1_w"] + params["lab1_b"])
             @ params["lab2_w"] + params["lab2_b"])
    return (pose, label)


# speedup vs baseline: 1.4920x; 1.1881x over previous
import functools

import jax
import jax.numpy as jnp
from jax import lax
from jax.experimental import pallas as pl
from jax.experimental.pallas import tpu as pltpu
from jax.experimental.pallas import tpu_sc as plsc

N = 10000
E = 320000
H = 128
HH = H // 2
K = 5

NCORES = 2
NSUB = 16
EPT = E // NSUB
CH2 = 128
NCH = 157
EPAD = NCH * CH2
EL = E + EPAD - EPT
NPAD = 10112
RPT = NPAD // NSUB
ZQ = 2
ZROWS = RPT // ZQ


def _make_segsum_body(table_half_rows, mode):
    off = table_half_rows

    def body(table, srcs, dsts, out, src_v, dst_v, src_cur0, dst_cur0,
             src_cur1, dst_cur1, rows0, rows1, zbuf, y_sh, gsem0, gsem1):
        c = lax.axis_index("c")
        s = lax.axis_index("s")
        coff = c.astype(jnp.int32) * off

        if mode == "gather":
            pltpu.sync_copy(srcs.at[s], src_v)
        pltpu.sync_copy(dsts.at[s], dst_v)

        @pl.loop(0, ZROWS)
        def _zero(i):
            for j in range(HH // 16):
                zbuf[i, pl.ds(j * 16, 16)] = jnp.zeros((16,), jnp.float32)

        for q in range(ZQ):
            pltpu.sync_copy(zbuf, y_sh.at[pl.ds(s * RPT + q * ZROWS, ZROWS)])

        if mode == "ones":
            @pl.loop(0, CH2)
            def _fill(i):
                for j in range(HH // 16):
                    rows0[i, pl.ds(j * 16, 16)] = (
                        jnp.zeros((16,), jnp.float32) + 1.0)

        plsc.subcore_barrier()

        def build_dst(j, dc):
            for i in range(CH2 // 16):
                dc[pl.ds(i * 16, 16)] = dst_v[j, pl.ds(i * 16, 16)]

        if mode == "ones":
            @pl.loop(0, NCH)
            def _edges(j):
                build_dst(j, dst_cur0)
                pltpu.sync_copy(rows0, y_sh.at[dst_cur0], add=True)
        else:
            lbase = (c * EL + s * EPT) if mode == "linear" else 0

            def fetch_src(j, sc):
                if mode == "linear":
                    return table.at[pl.ds(lbase + j * CH2, CH2)]
                return table.at[sc]

            def build_fire(j, sc, rows, sem):
                if mode == "gather":
                    for i in range(CH2 // 16):
                        sc[pl.ds(i * 16, 16)] = (
                            src_v[j, pl.ds(i * 16, 16)] + coff)
                pltpu.async_copy(fetch_src(j, sc), rows, sem)

            build_dst(0, dst_cur0)
            build_fire(0, src_cur0, rows0, gsem0)

            @pl.loop(0, NCH // 2)
            def _pairs(g):
                j0 = 2 * g
                build_dst(j0 + 1, dst_cur1)
                build_fire(j0 + 1, src_cur1, rows1, gsem1)
                pltpu.make_async_copy(fetch_src(j0, src_cur0),
                                      rows0, gsem0).wait()
                pltpu.sync_copy(rows0, y_sh.at[dst_cur0], add=True)
                build_dst(j0 + 2, dst_cur0)
                build_fire(j0 + 2, src_cur0, rows0, gsem0)
                pltpu.make_async_copy(fetch_src(j0 + 1, src_cur1),
                                      rows1, gsem1).wait()
                pltpu.sync_copy(rows1, y_sh.at[dst_cur1], add=True)

            jl = NCH - 1
            pltpu.make_async_copy(fetch_src(jl, src_cur0), rows0, gsem0).wait()
            pltpu.sync_copy(rows0, y_sh.at[dst_cur0], add=True)

        plsc.subcore_barrier()

        pltpu.sync_copy(y_sh.at[pl.ds(s * RPT, RPT)],
                        out.at[c, pl.ds(s * RPT, RPT)])

    return body


@functools.partial(jax.jit, static_argnums=(3, 4))
def _sc_segsum(table, srcs, dsts, table_half_rows, mode="gather"):
    mesh = plsc.VectorSubcoreMesh(core_axis_name="c", subcore_axis_name="s",
                                  num_cores=NCORES, num_subcores=NSUB)
    f = pl.kernel(
        _make_segsum_body(table_half_rows, mode),
        out_type=jax.ShapeDtypeStruct((NCORES, NPAD, HH), jnp.float32),
        mesh=mesh,
        scratch_types=[
            pltpu.VMEM((NCH, CH2), jnp.int32),
            pltpu.VMEM((NCH, CH2), jnp.int32),
            pltpu.VMEM((CH2,), jnp.int32),
            pltpu.VMEM((CH2,), jnp.int32),
            pltpu.VMEM((CH2,), jnp.int32),
            pltpu.VMEM((CH2,), jnp.int32),
            pltpu.VMEM((CH2, HH), jnp.float32),
            pltpu.VMEM((CH2, HH), jnp.float32),
            pltpu.VMEM((ZROWS, HH), jnp.float32),
            pltpu.VMEM_SHARED((NPAD, HH), jnp.float32),
            pltpu.SemaphoreType.DMA,
            pltpu.SemaphoreType.DMA,
        ],
        compiler_params=pltpu.CompilerParams(use_tc_tiling_on_sc=False),
    )
    return f(table, srcs, dsts)


def _pad_idx(idx, pad_value):
    idx = idx.reshape(NSUB, EPT)
    pad = jnp.full((NSUB, EPAD - EPT), pad_value, jnp.int32)
    return jnp.concatenate([idx, pad], axis=1).reshape(NSUB, NCH, CH2)


def _bn_relu(x, g, b):
    m = x.mean(axis=0)
    v = x.var(axis=0)
    return jax.nn.relu((x - m) / jnp.sqrt(v + 1e-5) * g + b)


def _tagconv(h, srcs, dsts, norm, n2pad, w, b):
    t = (h * norm).reshape(N, 2, HH).swapaxes(0, 1).reshape(2 * N, HH)
    acc = h @ w[:H]
    for k in range(1, K + 1):
        parts = _sc_segsum(t, srcs, dsts, N if k == 1 else NPAD)
        wk = w[k * H:(k + 1) * H]
        pk = norm * (parts[0, :N] @ wk[:HH] + parts[1, :N] @ wk[HH:])
        acc = acc + pk
        if k < K:
            t = (parts * n2pad).reshape(2 * NPAD, HH)
    return acc + b


def kernel(node_features, edge_index, edge_attr, lap_pe, params):
    src = edge_index[0]
    dst = edge_index[1]
    srcs = _pad_idx(src, 0)
    dsts = _pad_idx(dst, N)

    ones_tab = jnp.ones((16, HH), jnp.float32)
    deg = _sc_segsum(ones_tab, srcs, dsts, 8, "ones")
    deg = deg[0, :N, 0]
    norm = jnp.power(jnp.clip(deg, 1.0, None), -0.5)[:, None]
    n2pad = jnp.pad((norm * norm), ((0, NPAD - N), (0, 0)))[None]

    eproc = edge_attr @ params["edge_w"] + params["edge_b"]
    et = eproc.reshape(E, 2, HH).swapaxes(0, 1)
    et = jnp.pad(et, ((0, 0), (0, EL - E), (0, 0))).reshape(2 * EL, HH)
    agg = _sc_segsum(et, srcs, dsts, E, "linear")
    agg_edge = jnp.concatenate([agg[0, :N], agg[1, :N]], axis=1)

    h = (node_features @ params["in_w"] + params["in_b"]
         + lap_pe @ params["pos_w"] + params["pos_b"]
         + agg_edge)
    for m in params["blocks"]:
        h_in = h
        h = _tagconv(h, srcs, dsts, norm, n2pad, m["tag1_w"], m["tag1_b"])
        h = _bn_relu(h, m["bn1_g"], m["bn1_b"])
        h = _tagconv(h, srcs, dsts, norm, n2pad, m["tag2_w"], m["tag2_b"])
        h = _bn_relu(h, m["bn2_g"], m["bn2_b"])
        h = h @ m["ff_w"] + m["ff_b"]
        h = h + h_in
    pose = (jax.nn.relu(h @ params["pose1_w"] + params["pose1_b"])
            @ params["pose2_w"] + params["pose2_b"])
    y = h.mean(axis=0, keepdims=True)
    label = (jax.nn.relu(y @ params["lab1_w"] + params["lab1_b"])
             @ params["lab2_w"] + params["lab2_b"])
    return (pose, label)
